# baseline (speedup 1.0000x reference)
import functools

import jax
import jax.numpy as jnp
from jax import lax
from jax.experimental import pallas as pl
from jax.experimental.pallas import tpu as pltpu
from jax.experimental.pallas import tpu_sc as plsc

N_NODES = 10000
D = 128
N_EDGES = 320000
N_GRAPHS = 64

NC = 2
NS = 16
NW = NC * NS
E_PER_TILE = N_EDGES // NW
CHUNK = 125
NCHUNK = E_PER_TILE // CHUNK
NPAD = 10240
ROWS_PER_TILE = NPAD // NS
ZROWS = 128
HP = 10240
HSLC = HP // NS

_mesh = plsc.VectorSubcoreMesh(core_axis_name="c", subcore_axis_name="s")

BLK = 1024
NBLK = NPAD // BLK


def _sc_degree(dst2d):
    @functools.partial(
        pl.kernel,
        out_type=jax.ShapeDtypeStruct((NC * HP,), jnp.float32),
        mesh=_mesh,
        scratch_types=[
            pltpu.VMEM((NCHUNK, CHUNK), jnp.int32),
            pltpu.VMEM((128,), jnp.float32),
            pltpu.VMEM((HSLC,), jnp.float32),
            pltpu.VMEM_SHARED((HP,), jnp.float32),
        ],
    )
    def k(dst_hbm, out_hbm, didx, ones_v, zv, shist):
        cid = lax.axis_index("c")
        sid = lax.axis_index("s")
        wid = cid * NS + sid

        @pl.loop(0, 128, step=16)
        def _(i):
            ones_v[pl.ds(i, 16)] = jnp.full((16,), 1.0, jnp.float32)

        @pl.loop(0, HSLC, step=16)
        def _(i):
            zv[pl.ds(i, 16)] = jnp.zeros((16,), jnp.float32)

        pltpu.sync_copy(zv, shist.at[pl.ds(sid * HSLC, HSLC)])
        pltpu.sync_copy(dst_hbm.at[pl.ds(wid * NCHUNK, NCHUNK)], didx)
        plsc.subcore_barrier()

        @pl.loop(0, NCHUNK)
        def _(j):
            pltpu.sync_copy(ones_v.at[pl.ds(0, CHUNK)],
                            shist.at[didx.at[j]], add=True)

        plsc.subcore_barrier()
        pltpu.sync_copy(shist.at[pl.ds(sid * HSLC, HSLC)],
                        out_hbm.at[pl.ds(cid * HP + sid * HSLC, HSLC)])

    return k(dst2d)


GRP = 8
NGRP = NCHUNK // GRP

EPAD = 327680
DUMP = NPAD - 1
CHP = 80
NCHP = EPAD // NW // CHP
NGRPP = NCHP // GRP


def _sc_aggregate(y, srcp, dstp):
    @functools.partial(
        pl.kernel,
        out_type=jax.ShapeDtypeStruct((NC * NPAD, D), jnp.float32),
        mesh=_mesh,
        scratch_types=(
            [pltpu.VMEM((GRP, CHP), jnp.int32)] * 4
            + [pltpu.VMEM((GRP, CHP), jnp.int32)] * 4
            + [pltpu.VMEM((CHP, D), jnp.float32)] * 4
            + [pltpu.VMEM_SHARED((NPAD, D), jnp.float32)]
            + [pltpu.SemaphoreType.DMA] * 12
        ),
    )
    def k(y_hbm, src_hbm, dst_hbm, out_hbm,
          sr0, sr1, sr2, sr3, dr0, dr1, dr2, dr3,
          r0, r1, r2, r3, acc,
          i0, i1, i2, i3, g0, g1, g2, g3, s0, s1, s2, s3):
        cid = lax.axis_index("c")
        sid = lax.axis_index("s")
        wid = cid * NS + sid
        srings = (sr0, sr1, sr2, sr3)
        drings = (dr0, dr1, dr2, dr3)
        rows = (r0, r1, r2, r3)
        isems = (i0, i1, i2, i3)
        gsems = (g0, g1, g2, g3)
        ssems = (s0, s1, s2, s3)
        brow = wid * NCHP

        def idx_start(grp, sl):
            pltpu.async_copy(src_hbm.at[pl.ds(brow + grp * GRP, GRP)],
                             srings[sl], isems[sl])
            pltpu.async_copy(dst_hbm.at[pl.ds(brow + grp * GRP, GRP)],
                             drings[sl], isems[sl])

        def idx_wait(grp, sl):
            pltpu.make_async_copy(src_hbm.at[pl.ds(brow + grp * GRP, GRP)],
                                  srings[sl], isems[sl]).wait()
            pltpu.make_async_copy(dst_hbm.at[pl.ds(brow + grp * GRP, GRP)],
                                  drings[sl], isems[sl]).wait()

        def gather_start(sl, m, q):
            pltpu.async_copy(y_hbm.at[srings[sl].at[m]], rows[q], gsems[q])

        def gather_wait(sl, m, q):
            pltpu.make_async_copy(y_hbm.at[srings[sl].at[m]], rows[q],
                                  gsems[q]).wait()

        def scat_start(sl, m, q):
            pltpu.async_copy(rows[q], acc.at[drings[sl].at[m]], ssems[q],
                             add=True)

        def scat_wait(sl, m, q):
            pltpu.make_async_copy(rows[q], acc.at[drings[sl].at[m]],
                                  ssems[q]).wait()

        @pl.loop(0, CHP)
        def _(r):
            @pl.loop(0, D, step=16)
            def _(c):
                r0[r, pl.ds(c, 16)] = jnp.zeros((16,), jnp.float32)

        zbase = sid * ROWS_PER_TILE

        @pl.loop(0, ROWS_PER_TILE, step=CHP)
        def _(o):
            pltpu.async_copy(r0, acc.at[pl.ds(zbase + o, CHP)], g0)

        @pl.loop(0, ROWS_PER_TILE, step=CHP)
        def _(o):
            pltpu.make_async_copy(r0, acc.at[pl.ds(zbase + o, CHP)],
                                  g0).wait()

        plsc.subcore_barrier()

        idx_start(0, 0)
        idx_start(1, 1)
        idx_start(2, 2)
        idx_wait(0, 0)
        gather_start(0, 0, 0)
        gather_start(0, 1, 1)

        def chunk_body(G, sl, m):
            q = m % 4
            t = G * GRP + m
            gather_wait(sl, m, q)
            scat_start(sl, m, q)

            if m >= 2:
                pv_sl, pv_m = sl, m - 2
            else:
                pv_sl, pv_m = (sl + 3) % 4, m + GRP - 2

            @pl.when(t > 1)
            def _():
                scat_wait(pv_sl, pv_m, (q + 2) % 4)

            if m == 2:
                @pl.when(G + 3 < NGRPP)
                def _():
                    idx_start(G + 3, (sl + 3) % 4)

            if m < GRP - 2:
                gather_start(sl, m + 2, (q + 2) % 4)
            elif m == GRP - 2:
                @pl.when(G + 1 < NGRPP)
                def _():
                    idx_wait(G + 1, (sl + 1) % 4)
                    gather_start((sl + 1) % 4, 0, (q + 2) % 4)
            else:
                @pl.when(G + 1 < NGRPP)
                def _():
                    gather_start((sl + 1) % 4, 1, (q + 2) % 4)

        @pl.loop(0, NGRPP, step=4)
        def _(G):
            for k_ in range(4):
                for m in range(GRP):
                    chunk_body(G + k_, k_, m)

        scat_wait(3, GRP - 2, 2)
        scat_wait(3, GRP - 1, 3)
        plsc.subcore_barrier()

        @pl.loop(0, ROWS_PER_TILE, step=ZROWS)
        def _(o):
            pltpu.async_copy(acc.at[pl.ds(zbase + o, ZROWS)],
                             out_hbm.at[pl.ds(cid * NPAD + zbase + o, ZROWS)],
                             g0)

        @pl.loop(0, ROWS_PER_TILE, step=ZROWS)
        def _(o):
            pltpu.make_async_copy(acc.at[pl.ds(zbase + o, ZROWS)],
                                  out_hbm.at[pl.ds(cid * NPAD + zbase + o,
                                                   ZROWS)], g0).wait()

    return k(y, srcp, dstp)


def _tc_matmul(x, W):
    def body(x_ref, w_ref, o_ref):
        o_ref[...] = jnp.dot(x_ref[...], w_ref[...],
                             preferred_element_type=jnp.float32)

    return pl.pallas_call(
        body,
        grid=(NBLK,),
        in_specs=[pl.BlockSpec((BLK, D), lambda i: (i, 0)),
                  pl.BlockSpec((D, D), lambda i: (0, 0))],
        out_specs=pl.BlockSpec((BLK, D), lambda i: (i, 0)),
        out_shape=jax.ShapeDtypeStruct((NPAD, D), jnp.float32),
    )(x, W)


def _tc_scale(xw, dpt):
    def body(xw_ref, dp_ref, y_ref, dinv_ref):
        deg = 1.0 + dp_ref[:, 0:1] + dp_ref[:, 1:2]
        dinv = lax.rsqrt(deg)
        dinv_ref[...] = dinv
        y_ref[...] = xw_ref[...] * dinv

    return pl.pallas_call(
        body,
        grid=(NBLK,),
        in_specs=[pl.BlockSpec((BLK, D), lambda i: (i, 0)),
                  pl.BlockSpec((BLK, 2), lambda i: (i, 0))],
        out_specs=[pl.BlockSpec((BLK, D), lambda i: (i, 0)),
                   pl.BlockSpec((BLK, 1), lambda i: (i, 0))],
        out_shape=[jax.ShapeDtypeStruct((NPAD, D), jnp.float32),
                   jax.ShapeDtypeStruct((NPAD, 1), jnp.float32)],
    )(xw, dpt)


_A0 = pl.BlockSpec((BLK, D), lambda i: (i, 0))
_A1 = pl.BlockSpec((BLK, D), lambda i: (NBLK + i, 0))


def _tc_mid(agg, y1, dinv, b1, W2):
    def body(a0_ref, a1_ref, y1_ref, dinv_ref, b1_ref, w2_ref, y2_ref):
        dinv = dinv_ref[...]
        h = (a0_ref[...] + a1_ref[...] + y1_ref[...]) * dinv + b1_ref[...]
        h = jnp.maximum(h, 0.0)
        y2_ref[...] = jnp.dot(h, w2_ref[...],
                              preferred_element_type=jnp.float32) * dinv

    return pl.pallas_call(
        body,
        grid=(NBLK,),
        in_specs=[_A0, _A1,
                  pl.BlockSpec((BLK, D), lambda i: (i, 0)),
                  pl.BlockSpec((BLK, 1), lambda i: (i, 0)),
                  pl.BlockSpec((1, D), lambda i: (0, 0)),
                  pl.BlockSpec((D, D), lambda i: (0, 0))],
        out_specs=pl.BlockSpec((BLK, D), lambda i: (i, 0)),
        out_shape=jax.ShapeDtypeStruct((NPAD, D), jnp.float32),
    )(agg, agg, y1, dinv, b1, W2)


def _tc_pool(agg, y2, dinv, b2, bcol):
    def body(a0_ref, a1_ref, y2_ref, dinv_ref, b2_ref, b_ref, p_ref):
        i = pl.program_id(0)

        @pl.when(i == 0)
        def _():
            p_ref[...] = jnp.full((N_GRAPHS, D), -jnp.inf, jnp.float32)

        h = ((a0_ref[...] + a1_ref[...] + y2_ref[...]) * dinv_ref[...]
             + b2_ref[...])
        b = b_ref[...]
        lo = jnp.maximum(jnp.min(b), 0)
        hi = jnp.max(b)

        def upd(g, carry):
            m = jnp.max(jnp.where(b == g, h, -jnp.inf), axis=0, keepdims=True)
            p_ref[pl.ds(g, 1), :] = jnp.maximum(p_ref[pl.ds(g, 1), :], m)
            return carry

        lax.fori_loop(lo, hi + 1, upd, 0)

    return pl.pallas_call(
        body,
        grid=(NBLK,),
        in_specs=[_A0, _A1,
                  pl.BlockSpec((BLK, D), lambda i: (i, 0)),
                  pl.BlockSpec((BLK, 1), lambda i: (i, 0)),
                  pl.BlockSpec((1, D), lambda i: (0, 0)),
                  pl.BlockSpec((BLK, 1), lambda i: (i, 0))],
        out_specs=pl.BlockSpec((N_GRAPHS, D), lambda i: (0, 0)),
        out_shape=jax.ShapeDtypeStruct((N_GRAPHS, D), jnp.float32),
    )(agg, agg, y2, dinv, b2, bcol)


def _tc_dec(pooled, Wd, bd):
    NP = Wd.shape[1]
    CBLK = 1024

    def body(p_ref, wd_ref, bd_ref, o_ref):
        o_ref[...] = jnp.dot(p_ref[...], wd_ref[...],
                             preferred_element_type=jnp.float32) + bd_ref[...]

    return pl.pallas_call(
        body,
        grid=(NP // CBLK,),
        in_specs=[pl.BlockSpec((N_GRAPHS, D), lambda i: (0, 0)),
                  pl.BlockSpec((D, CBLK), lambda i: (0, i)),
                  pl.BlockSpec((1, CBLK), lambda i: (0, i))],
        out_specs=pl.BlockSpec((N_GRAPHS, CBLK), lambda i: (0, i)),
        out_shape=jax.ShapeDtypeStruct((N_GRAPHS, NP), jnp.float32),
    )(pooled, Wd, bd)


def kernel(x, edge_index, batch, W1, b1, W2, b2, Wd, bd):
    dst2d = edge_index[1].reshape(N_EDGES // CHUNK, CHUNK)
    srcp = jnp.concatenate(
        [edge_index[0], jnp.zeros((EPAD - N_EDGES,), jnp.int32)]
    ).reshape(EPAD // CHP, CHP)
    dstp = jnp.concatenate(
        [edge_index[1], jnp.full((EPAD - N_EDGES,), DUMP, jnp.int32)]
    ).reshape(EPAD // CHP, CHP)

    x_p = jnp.pad(x, ((0, NPAD - N_NODES), (0, 0)))
    b_p = jnp.pad(batch.reshape(N_NODES, 1), ((0, NPAD - N_NODES), (0, 0)),
                  constant_values=-1)

    degp = _sc_degree(dst2d).reshape(NC, HP)
    xw1 = _tc_matmul(x_p, W1)
    dpt = jnp.transpose(degp)
    y1, dinv = _tc_scale(xw1, dpt)

    agg1 = _sc_aggregate(y1, srcp, dstp)
    y2 = _tc_mid(agg1, y1, dinv, b1.reshape(1, D), W2)

    agg2 = _sc_aggregate(y2, srcp, dstp)
    pooled = _tc_pool(agg2, y2, dinv, b2.reshape(1, D), b_p)

    Wd_p = jnp.pad(Wd, ((0, 0), (0, NPAD - N_NODES)))
    bd_p = jnp.pad(bd.reshape(1, N_NODES), ((0, 0), (0, NPAD - N_NODES)))
    return _tc_dec(pooled, Wd_p, bd_p)[:, :N_NODES]


# speedup vs baseline: 3.2428x; 3.2428x over previous
import functools

import jax
import jax.numpy as jnp
from jax import lax
from jax.experimental import pallas as pl
from jax.experimental.pallas import tpu as pltpu
from jax.experimental.pallas import tpu_sc as plsc

N_NODES = 10000
D = 128
N_EDGES = 320000
N_GRAPHS = 64

NC = 2
NS = 16
NW = NC * NS
E_PER_TILE = N_EDGES // NW
CHUNK = 125
NCHUNK = E_PER_TILE // CHUNK
NPAD = 10240
ROWS_PER_TILE = NPAD // NS
ZROWS = 128
HP = 10240
HSLC = HP // NS

_mesh = plsc.VectorSubcoreMesh(core_axis_name="c", subcore_axis_name="s")

BLK = 1024
NBLK = NPAD // BLK


def _sc_degree(dst2d):
    @functools.partial(
        pl.kernel,
        out_type=jax.ShapeDtypeStruct((NC * HP,), jnp.float32),
        mesh=_mesh,
        scratch_types=[
            pltpu.VMEM((NCHUNK, CHUNK), jnp.int32),
            pltpu.VMEM((128,), jnp.float32),
            pltpu.VMEM((HSLC,), jnp.float32),
            pltpu.VMEM_SHARED((HP,), jnp.float32),
        ],
    )
    def k(dst_hbm, out_hbm, didx, ones_v, zv, shist):
        cid = lax.axis_index("c")
        sid = lax.axis_index("s")
        wid = cid * NS + sid

        @pl.loop(0, 128, step=16)
        def _(i):
            ones_v[pl.ds(i, 16)] = jnp.full((16,), 1.0, jnp.float32)

        @pl.loop(0, HSLC, step=16)
        def _(i):
            zv[pl.ds(i, 16)] = jnp.zeros((16,), jnp.float32)

        pltpu.sync_copy(zv, shist.at[pl.ds(sid * HSLC, HSLC)])
        pltpu.sync_copy(dst_hbm.at[pl.ds(wid * NCHUNK, NCHUNK)], didx)
        plsc.subcore_barrier()

        @pl.loop(0, NCHUNK)
        def _(j):
            pltpu.sync_copy(ones_v.at[pl.ds(0, CHUNK)],
                            shist.at[didx.at[j]], add=True)

        plsc.subcore_barrier()
        pltpu.sync_copy(shist.at[pl.ds(sid * HSLC, HSLC)],
                        out_hbm.at[pl.ds(cid * HP + sid * HSLC, HSLC)])

    return k(dst2d)


GRP = 8
NGRP = NCHUNK // GRP


def _sc_aggregate(y, src2d, dst2d):
    @functools.partial(
        pl.kernel,
        out_type=jax.ShapeDtypeStruct((NC * NPAD, D), jnp.float32),
        mesh=_mesh,
        scratch_types=[
            pltpu.VMEM((GRP, CHUNK), jnp.int32),
            pltpu.VMEM((GRP, CHUNK), jnp.int32),
            pltpu.VMEM((GRP, CHUNK), jnp.int32),
            pltpu.VMEM((GRP, CHUNK), jnp.int32),
            pltpu.VMEM((CHUNK, D), jnp.float32),
            pltpu.VMEM((CHUNK, D), jnp.float32),
            pltpu.VMEM_SHARED((NPAD, D), jnp.float32),
            pltpu.SemaphoreType.DMA,
            pltpu.SemaphoreType.DMA,
            pltpu.SemaphoreType.DMA,
            pltpu.SemaphoreType.DMA,
            pltpu.SemaphoreType.DMA,
            pltpu.SemaphoreType.DMA,
        ],
    )
    def k(y_hbm, src_hbm, dst_hbm, out_hbm,
          sr0, sr1, dr0, dr1, rows0, rows1, acc,
          is0, is1, gs0, gs1, ss0, ss1):
        cid = lax.axis_index("c")
        sid = lax.axis_index("s")
        wid = cid * NS + sid
        srings, drings = (sr0, sr1), (dr0, dr1)
        rows = (rows0, rows1)
        gsems, isems, ssems = (gs0, gs1), (is0, is1), (ss0, ss1)
        brow = wid * NCHUNK

        def idx_start(grp, s):
            pltpu.async_copy(src_hbm.at[pl.ds(brow + grp * GRP, GRP)],
                             srings[s], isems[s])
            pltpu.async_copy(dst_hbm.at[pl.ds(brow + grp * GRP, GRP)],
                             drings[s], isems[s])

        def idx_wait(grp, s):
            pltpu.make_async_copy(src_hbm.at[pl.ds(brow + grp * GRP, GRP)],
                                  srings[s], isems[s]).wait()
            pltpu.make_async_copy(dst_hbm.at[pl.ds(brow + grp * GRP, GRP)],
                                  drings[s], isems[s]).wait()

        def gather_start(s, m, rb):
            pltpu.async_copy(y_hbm.at[srings[s].at[m]], rows[rb], gsems[rb])

        def gather_wait(s, m, rb):
            pltpu.make_async_copy(y_hbm.at[srings[s].at[m]], rows[rb],
                                  gsems[rb]).wait()

        def scat_start(s, m, rb):
            pltpu.async_copy(rows[rb], acc.at[drings[s].at[m]], ssems[rb],
                             add=True)

        def scat_wait(s, m, rb):
            pltpu.make_async_copy(rows[rb], acc.at[drings[s].at[m]],
                                  ssems[rb]).wait()

        @pl.loop(0, CHUNK)
        def _(r):
            @pl.loop(0, D, step=16)
            def _(c):
                rows0[r, pl.ds(c, 16)] = jnp.zeros((16,), jnp.float32)

        zbase = sid * ROWS_PER_TILE

        @pl.loop(0, 5 * CHUNK, step=CHUNK)
        def _(o):
            pltpu.async_copy(rows0, acc.at[pl.ds(zbase + o, CHUNK)], gs0)

        pltpu.async_copy(rows0.at[pl.ds(0, ROWS_PER_TILE - 5 * CHUNK)],
                         acc.at[pl.ds(zbase + 5 * CHUNK,
                                      ROWS_PER_TILE - 5 * CHUNK)], gs1)

        @pl.loop(0, 5 * CHUNK, step=CHUNK)
        def _(o):
            pltpu.make_async_copy(rows0, acc.at[pl.ds(zbase + o, CHUNK)],
                                  gs0).wait()

        pltpu.make_async_copy(rows0.at[pl.ds(0, ROWS_PER_TILE - 5 * CHUNK)],
                              acc.at[pl.ds(zbase + 5 * CHUNK,
                                           ROWS_PER_TILE - 5 * CHUNK)],
                              gs1).wait()
        plsc.subcore_barrier()

        idx_start(0, 0)
        idx_wait(0, 0)
        gather_start(0, 0, 0)

        def pair_body(g, s, m):
            j = g * GRP + m
            gather_wait(s, m, 0)
            scat_start(s, m, 0)

            @pl.when(j > 0)
            def _():
                pltpu.make_async_copy(rows1, acc.at[dr0.at[0]], ss1).wait()

            @pl.when(jnp.logical_and(m == 2, g + 1 < NGRP))
            def _():
                idx_start(g + 1, 1 - s)

            gather_start(s, m + 1, 1)

            gather_wait(s, m + 1, 1)
            scat_start(s, m + 1, 1)
            pltpu.make_async_copy(rows0, acc.at[dr0.at[0]], ss0).wait()

            @pl.when(m < GRP - 2)
            def _():
                gather_start(s, m + 2, 0)

            @pl.when(jnp.logical_and(m == GRP - 2, g + 1 < NGRP))
            def _():
                idx_wait(g + 1, 1 - s)
                gather_start(1 - s, 0, 0)

        def group_body(g, s):
            @pl.loop(0, GRP, step=2)
            def _(m):
                pair_body(g, s, m)

        @pl.loop(0, NGRP, step=2)
        def _(g):
            group_body(g, 0)
            group_body(g + 1, 1)

        pltpu.make_async_copy(rows1, acc.at[dr0.at[0]], ss1).wait()
        plsc.subcore_barrier()

        @pl.loop(0, ROWS_PER_TILE, step=ZROWS)
        def _(o):
            pltpu.async_copy(acc.at[pl.ds(zbase + o, ZROWS)],
                             out_hbm.at[pl.ds(cid * NPAD + zbase + o, ZROWS)],
                             gs0)

        @pl.loop(0, ROWS_PER_TILE, step=ZROWS)
        def _(o):
            pltpu.make_async_copy(acc.at[pl.ds(zbase + o, ZROWS)],
                                  out_hbm.at[pl.ds(cid * NPAD + zbase + o,
                                                   ZROWS)], gs0).wait()

    return k(y, src2d, dst2d)


def _tc_matmul(x, W):
    def body(x_ref, w_ref, o_ref):
        o_ref[...] = jnp.dot(x_ref[...], w_ref[...],
                             preferred_element_type=jnp.float32)

    return pl.pallas_call(
        body,
        grid=(NBLK,),
        in_specs=[pl.BlockSpec((BLK, D), lambda i: (i, 0)),
                  pl.BlockSpec((D, D), lambda i: (0, 0))],
        out_specs=pl.BlockSpec((BLK, D), lambda i: (i, 0)),
        out_shape=jax.ShapeDtypeStruct((NPAD, D), jnp.float32),
    )(x, W)


def _tc_scale(xw, dpt):
    def body(xw_ref, dp_ref, y_ref, dinv_ref):
        deg = 1.0 + dp_ref[:, 0:1] + dp_ref[:, 1:2]
        dinv = lax.rsqrt(deg)
        dinv_ref[...] = dinv
        y_ref[...] = xw_ref[...] * dinv

    return pl.pallas_call(
        body,
        grid=(NBLK,),
        in_specs=[pl.BlockSpec((BLK, D), lambda i: (i, 0)),
                  pl.BlockSpec((BLK, 2), lambda i: (i, 0))],
        out_specs=[pl.BlockSpec((BLK, D), lambda i: (i, 0)),
                   pl.BlockSpec((BLK, 1), lambda i: (i, 0))],
        out_shape=[jax.ShapeDtypeStruct((NPAD, D), jnp.float32),
                   jax.ShapeDtypeStruct((NPAD, 1), jnp.float32)],
    )(xw, dpt)


_A0 = pl.BlockSpec((BLK, D), lambda i: (i, 0))
_A1 = pl.BlockSpec((BLK, D), lambda i: (NBLK + i, 0))


def _tc_mid(agg, y1, dinv, b1, W2):
    def body(a0_ref, a1_ref, y1_ref, dinv_ref, b1_ref, w2_ref, y2_ref):
        dinv = dinv_ref[...]
        h = (a0_ref[...] + a1_ref[...] + y1_ref[...]) * dinv + b1_ref[...]
        h = jnp.maximum(h, 0.0)
        y2_ref[...] = jnp.dot(h, w2_ref[...],
                              preferred_element_type=jnp.float32) * dinv

    return pl.pallas_call(
        body,
        grid=(NBLK,),
        in_specs=[_A0, _A1,
                  pl.BlockSpec((BLK, D), lambda i: (i, 0)),
                  pl.BlockSpec((BLK, 1), lambda i: (i, 0)),
                  pl.BlockSpec((1, D), lambda i: (0, 0)),
                  pl.BlockSpec((D, D), lambda i: (0, 0))],
        out_specs=pl.BlockSpec((BLK, D), lambda i: (i, 0)),
        out_shape=jax.ShapeDtypeStruct((NPAD, D), jnp.float32),
    )(agg, agg, y1, dinv, b1, W2)


def _tc_pool(agg, y2, dinv, b2, bcol):
    def body(a0_ref, a1_ref, y2_ref, dinv_ref, b2_ref, b_ref, p_ref):
        i = pl.program_id(0)

        @pl.when(i == 0)
        def _():
            p_ref[...] = jnp.full((N_GRAPHS, D), -jnp.inf, jnp.float32)

        h = ((a0_ref[...] + a1_ref[...] + y2_ref[...]) * dinv_ref[...]
             + b2_ref[...])
        b = b_ref[...]
        lo = jnp.maximum(jnp.min(b), 0)
        hi = jnp.max(b)

        def upd(g, carry):
            m = jnp.max(jnp.where(b == g, h, -jnp.inf), axis=0, keepdims=True)
            p_ref[pl.ds(g, 1), :] = jnp.maximum(p_ref[pl.ds(g, 1), :], m)
            return carry

        lax.fori_loop(lo, hi + 1, upd, 0)

    return pl.pallas_call(
        body,
        grid=(NBLK,),
        in_specs=[_A0, _A1,
                  pl.BlockSpec((BLK, D), lambda i: (i, 0)),
                  pl.BlockSpec((BLK, 1), lambda i: (i, 0)),
                  pl.BlockSpec((1, D), lambda i: (0, 0)),
                  pl.BlockSpec((BLK, 1), lambda i: (i, 0))],
        out_specs=pl.BlockSpec((N_GRAPHS, D), lambda i: (0, 0)),
        out_shape=jax.ShapeDtypeStruct((N_GRAPHS, D), jnp.float32),
    )(agg, agg, y2, dinv, b2, bcol)


def _tc_dec(pooled, Wd, bd):
    NP = Wd.shape[1]
    CBLK = 1024

    def body(p_ref, wd_ref, bd_ref, o_ref):
        o_ref[...] = jnp.dot(p_ref[...], wd_ref[...],
                             preferred_element_type=jnp.float32) + bd_ref[...]

    return pl.pallas_call(
        body,
        grid=(NP // CBLK,),
        in_specs=[pl.BlockSpec((N_GRAPHS, D), lambda i: (0, 0)),
                  pl.BlockSpec((D, CBLK), lambda i: (0, i)),
                  pl.BlockSpec((1, CBLK), lambda i: (0, i))],
        out_specs=pl.BlockSpec((N_GRAPHS, CBLK), lambda i: (0, i)),
        out_shape=jax.ShapeDtypeStruct((N_GRAPHS, NP), jnp.float32),
    )(pooled, Wd, bd)


def kernel(x, edge_index, batch, W1, b1, W2, b2, Wd, bd):
    src2d = edge_index[0].reshape(N_EDGES // CHUNK, CHUNK)
    dst2d = edge_index[1].reshape(N_EDGES // CHUNK, CHUNK)

    x_p = jnp.pad(x, ((0, NPAD - N_NODES), (0, 0)))
    b_p = jnp.pad(batch.reshape(N_NODES, 1), ((0, NPAD - N_NODES), (0, 0)),
                  constant_values=-1)

    degp = _sc_degree(dst2d).reshape(NC, HP)
    xw1 = _tc_matmul(x_p, W1)
    dpt = jnp.transpose(degp)
    y1, dinv = _tc_scale(xw1, dpt)

    agg1 = _sc_aggregate(y1, src2d, dst2d)
    y2 = _tc_mid(agg1, y1, dinv, b1.reshape(1, D), W2)

    agg2 = _sc_aggregate(y2, src2d, dst2d)
    pooled = _tc_pool(agg2, y2, dinv, b2.reshape(1, D), b_p)

    Wd_p = jnp.pad(Wd, ((0, 0), (0, NPAD - N_NODES)))
    bd_p = jnp.pad(bd.reshape(1, N_NODES), ((0, 0), (0, NPAD - N_NODES)))
    return _tc_dec(pooled, Wd_p, bd_p)[:, :N_NODES]


# speedup vs baseline: 3.4450x; 1.0624x over previous
import functools

import jax
import jax.numpy as jnp
from jax import lax
from jax.experimental import pallas as pl
from jax.experimental.pallas import tpu as pltpu
from jax.experimental.pallas import tpu_sc as plsc

N_NODES = 10000
D = 128
N_EDGES = 320000
N_GRAPHS = 64

NC = 2
NS = 16
NW = NC * NS
E_PER_TILE = N_EDGES // NW
CHUNK = 125
NCHUNK = E_PER_TILE // CHUNK
NPAD = 10240
ROWS_PER_TILE = NPAD // NS
ZROWS = 128
HP = 10240
HSLC = HP // NS

_mesh = plsc.VectorSubcoreMesh(core_axis_name="c", subcore_axis_name="s")

BLK = 1024
NBLK = NPAD // BLK


def _sc_degree(dst2d):
    @functools.partial(
        pl.kernel,
        out_type=jax.ShapeDtypeStruct((NC * HP,), jnp.float32),
        mesh=_mesh,
        scratch_types=[
            pltpu.VMEM((NCHUNK, CHUNK), jnp.int32),
            pltpu.VMEM((128,), jnp.float32),
            pltpu.VMEM((HSLC,), jnp.float32),
            pltpu.VMEM_SHARED((HP,), jnp.float32),
        ],
    )
    def k(dst_hbm, out_hbm, didx, ones_v, zv, shist):
        cid = lax.axis_index("c")
        sid = lax.axis_index("s")
        wid = cid * NS + sid

        @pl.loop(0, 128, step=16)
        def _(i):
            ones_v[pl.ds(i, 16)] = jnp.full((16,), 1.0, jnp.float32)

        @pl.loop(0, HSLC, step=16)
        def _(i):
            zv[pl.ds(i, 16)] = jnp.zeros((16,), jnp.float32)

        pltpu.sync_copy(zv, shist.at[pl.ds(sid * HSLC, HSLC)])
        pltpu.sync_copy(dst_hbm.at[pl.ds(wid * NCHUNK, NCHUNK)], didx)
        plsc.subcore_barrier()

        @pl.loop(0, NCHUNK)
        def _(j):
            pltpu.sync_copy(ones_v.at[pl.ds(0, CHUNK)],
                            shist.at[didx.at[j]], add=True)

        plsc.subcore_barrier()
        pltpu.sync_copy(shist.at[pl.ds(sid * HSLC, HSLC)],
                        out_hbm.at[pl.ds(cid * HP + sid * HSLC, HSLC)])

    return k(dst2d)


GRP = 8
NGRP = NCHUNK // GRP

EPAD = 327680
CHP = 80
NCHP = EPAD // NW // CHP
NGRPP = NCHP // GRP


def _sc_aggregate(y, srcp, dstp):
    @functools.partial(
        pl.kernel,
        out_type=jax.ShapeDtypeStruct((NC * NPAD, D), jnp.float32),
        mesh=_mesh,
        scratch_types=(
            [pltpu.VMEM((GRP, CHP), jnp.int32)] * 4
            + [pltpu.VMEM((GRP, CHP), jnp.int32)] * 4
            + [pltpu.VMEM((CHP, D), jnp.float32)] * 4
            + [pltpu.VMEM_SHARED((NPAD, D), jnp.float32)]
            + [pltpu.SemaphoreType.DMA] * 12
        ),
    )
    def k(y_hbm, src_hbm, dst_hbm, out_hbm,
          sr0, sr1, sr2, sr3, dr0, dr1, dr2, dr3,
          r0, r1, r2, r3, acc,
          i0, i1, i2, i3, g0, g1, g2, g3, s0, s1, s2, s3):
        cid = lax.axis_index("c")
        sid = lax.axis_index("s")
        wid = cid * NS + sid
        srings = (sr0, sr1, sr2, sr3)
        drings = (dr0, dr1, dr2, dr3)
        rows = (r0, r1, r2, r3)
        isems = (i0, i1, i2, i3)
        gsems = (g0, g1, g2, g3)
        ssems = (s0, s1, s2, s3)
        brow = wid * NCHP

        def idx_start(grp, sl):
            pltpu.async_copy(src_hbm.at[pl.ds(brow + grp * GRP, GRP)],
                             srings[sl], isems[sl])
            pltpu.async_copy(dst_hbm.at[pl.ds(brow + grp * GRP, GRP)],
                             drings[sl], isems[sl])

        def idx_wait(grp, sl):
            pltpu.make_async_copy(src_hbm.at[pl.ds(brow + grp * GRP, GRP)],
                                  srings[sl], isems[sl]).wait()
            pltpu.make_async_copy(dst_hbm.at[pl.ds(brow + grp * GRP, GRP)],
                                  drings[sl], isems[sl]).wait()

        def gather_start(sl, m, q):
            pltpu.async_copy(y_hbm.at[srings[sl].at[m]], rows[q], gsems[q])

        def gather_wait(sl, m, q):
            pltpu.make_async_copy(y_hbm.at[srings[sl].at[m]], rows[q],
                                  gsems[q]).wait()

        def scat_start(sl, m, q):
            pltpu.async_copy(rows[q], acc.at[drings[sl].at[m]], ssems[q],
                             add=True)

        def scat_wait(sl, m, q):
            pltpu.make_async_copy(rows[q], acc.at[drings[sl].at[m]],
                                  ssems[q]).wait()

        @pl.loop(0, CHP)
        def _(r):
            @pl.loop(0, D, step=16)
            def _(c):
                r0[r, pl.ds(c, 16)] = jnp.zeros((16,), jnp.float32)

        zbase = sid * ROWS_PER_TILE

        @pl.loop(0, ROWS_PER_TILE, step=CHP)
        def _(o):
            pltpu.async_copy(r0, acc.at[pl.ds(zbase + o, CHP)], g0)

        @pl.loop(0, ROWS_PER_TILE, step=CHP)
        def _(o):
            pltpu.make_async_copy(r0, acc.at[pl.ds(zbase + o, CHP)],
                                  g0).wait()

        plsc.subcore_barrier()

        idx_start(0, 0)
        idx_start(1, 1)
        idx_start(2, 2)
        idx_wait(0, 0)
        gather_start(0, 0, 0)
        gather_start(0, 1, 1)

        def chunk_body(G, sl, m):
            q = m % 4
            t = G * GRP + m
            gather_wait(sl, m, q)
            scat_start(sl, m, q)

            if m >= 2:
                pv_sl, pv_m = sl, m - 2
            else:
                pv_sl, pv_m = (sl + 3) % 4, m + GRP - 2

            @pl.when(t > 1)
            def _():
                scat_wait(pv_sl, pv_m, (q + 2) % 4)

            if m == 2:
                @pl.when(G + 3 < NGRPP)
                def _():
                    idx_start(G + 3, (sl + 3) % 4)

            if m < GRP - 2:
                gather_start(sl, m + 2, (q + 2) % 4)
            elif m == GRP - 2:
                @pl.when(G + 1 < NGRPP)
                def _():
                    idx_wait(G + 1, (sl + 1) % 4)
                    gather_start((sl + 1) % 4, 0, (q + 2) % 4)
            else:
                @pl.when(G + 1 < NGRPP)
                def _():
                    gather_start((sl + 1) % 4, 1, (q + 2) % 4)

        @pl.loop(0, NGRPP, step=4)
        def _(G):
            for k_ in range(4):
                for m in range(GRP):
                    chunk_body(G + k_, k_, m)

        scat_wait(3, GRP - 2, 2)
        scat_wait(3, GRP - 1, 3)
        plsc.subcore_barrier()

        @pl.loop(0, ROWS_PER_TILE, step=ZROWS)
        def _(o):
            pltpu.async_copy(acc.at[pl.ds(zbase + o, ZROWS)],
                             out_hbm.at[pl.ds(cid * NPAD + zbase + o, ZROWS)],
                             g0)

        @pl.loop(0, ROWS_PER_TILE, step=ZROWS)
        def _(o):
            pltpu.make_async_copy(acc.at[pl.ds(zbase + o, ZROWS)],
                                  out_hbm.at[pl.ds(cid * NPAD + zbase + o,
                                                   ZROWS)], g0).wait()

    return k(y, srcp, dstp)


def _tc_matmul(x, W):
    def body(x_ref, w_ref, o_ref):
        o_ref[...] = jnp.dot(x_ref[...], w_ref[...],
                             preferred_element_type=jnp.float32)

    return pl.pallas_call(
        body,
        grid=(NBLK,),
        in_specs=[pl.BlockSpec((BLK, D), lambda i: (i, 0)),
                  pl.BlockSpec((D, D), lambda i: (0, 0))],
        out_specs=pl.BlockSpec((BLK, D), lambda i: (i, 0)),
        out_shape=jax.ShapeDtypeStruct((NPAD, D), jnp.float32),
    )(x, W)


def _tc_scale(xw, dpt):
    def body(xw_ref, dp_ref, y_ref, dinv_ref):
        deg = 1.0 + dp_ref[:, 0:1] + dp_ref[:, 1:2]
        dinv = lax.rsqrt(deg)
        dinv_ref[...] = dinv
        y_ref[...] = xw_ref[...] * dinv

    return pl.pallas_call(
        body,
        grid=(NBLK,),
        in_specs=[pl.BlockSpec((BLK, D), lambda i: (i, 0)),
                  pl.BlockSpec((BLK, 2), lambda i: (i, 0))],
        out_specs=[pl.BlockSpec((BLK, D), lambda i: (i, 0)),
                   pl.BlockSpec((BLK, 1), lambda i: (i, 0))],
        out_shape=[jax.ShapeDtypeStruct((NPAD, D), jnp.float32),
                   jax.ShapeDtypeStruct((NPAD, 1), jnp.float32)],
    )(xw, dpt)


_A0 = pl.BlockSpec((BLK, D), lambda i: (i, 0))
_A1 = pl.BlockSpec((BLK, D), lambda i: (NBLK + i, 0))


def _tc_mid(agg, y1, dinv, b1, W2):
    def body(a0_ref, a1_ref, y1_ref, dinv_ref, b1_ref, w2_ref, y2_ref):
        dinv = dinv_ref[...]
        h = (a0_ref[...] + a1_ref[...] + y1_ref[...]) * dinv + b1_ref[...]
        h = jnp.maximum(h, 0.0)
        y2_ref[...] = jnp.dot(h, w2_ref[...],
                              preferred_element_type=jnp.float32) * dinv

    return pl.pallas_call(
        body,
        grid=(NBLK,),
        in_specs=[_A0, _A1,
                  pl.BlockSpec((BLK, D), lambda i: (i, 0)),
                  pl.BlockSpec((BLK, 1), lambda i: (i, 0)),
                  pl.BlockSpec((1, D), lambda i: (0, 0)),
                  pl.BlockSpec((D, D), lambda i: (0, 0))],
        out_specs=pl.BlockSpec((BLK, D), lambda i: (i, 0)),
        out_shape=jax.ShapeDtypeStruct((NPAD, D), jnp.float32),
    )(agg, agg, y1, dinv, b1, W2)


def _tc_pool(agg, y2, dinv, b2, bcol):
    def body(a0_ref, a1_ref, y2_ref, dinv_ref, b2_ref, b_ref, p_ref):
        i = pl.program_id(0)

        @pl.when(i == 0)
        def _():
            p_ref[...] = jnp.full((N_GRAPHS, D), -jnp.inf, jnp.float32)

        h = ((a0_ref[...] + a1_ref[...] + y2_ref[...]) * dinv_ref[...]
             + b2_ref[...])
        b = b_ref[...]
        lo = jnp.maximum(jnp.min(b), 0)
        hi = jnp.max(b)

        def upd(g, carry):
            m = jnp.max(jnp.where(b == g, h, -jnp.inf), axis=0, keepdims=True)
            p_ref[pl.ds(g, 1), :] = jnp.maximum(p_ref[pl.ds(g, 1), :], m)
            return carry

        lax.fori_loop(lo, hi + 1, upd, 0)

    return pl.pallas_call(
        body,
        grid=(NBLK,),
        in_specs=[_A0, _A1,
                  pl.BlockSpec((BLK, D), lambda i: (i, 0)),
                  pl.BlockSpec((BLK, 1), lambda i: (i, 0)),
                  pl.BlockSpec((1, D), lambda i: (0, 0)),
                  pl.BlockSpec((BLK, 1), lambda i: (i, 0))],
        out_specs=pl.BlockSpec((N_GRAPHS, D), lambda i: (0, 0)),
        out_shape=jax.ShapeDtypeStruct((N_GRAPHS, D), jnp.float32),
    )(agg, agg, y2, dinv, b2, bcol)


def _tc_dec(pooled, Wd, bd):
    NP = Wd.shape[1]
    CBLK = 1024

    def body(p_ref, wd_ref, bd_ref, o_ref):
        o_ref[...] = jnp.dot(p_ref[...], wd_ref[...],
                             preferred_element_type=jnp.float32) + bd_ref[...]

    return pl.pallas_call(
        body,
        grid=(NP // CBLK,),
        in_specs=[pl.BlockSpec((N_GRAPHS, D), lambda i: (0, 0)),
                  pl.BlockSpec((D, CBLK), lambda i: (0, i)),
                  pl.BlockSpec((1, CBLK), lambda i: (0, i))],
        out_specs=pl.BlockSpec((N_GRAPHS, CBLK), lambda i: (0, i)),
        out_shape=jax.ShapeDtypeStruct((N_GRAPHS, NP), jnp.float32),
    )(pooled, Wd, bd)


def kernel(x, edge_index, batch, W1, b1, W2, b2, Wd, bd):
    dst2d = edge_index[1].reshape(N_EDGES // CHUNK, CHUNK)
    pad_i = jnp.arange(EPAD - N_EDGES, dtype=jnp.int32)
    srcp = jnp.concatenate(
        [edge_index[0], pad_i % N_NODES]).reshape(EPAD // CHP, CHP)
    dstp = jnp.concatenate(
        [edge_index[1], N_NODES + pad_i % (NPAD - N_NODES)]
    ).reshape(EPAD // CHP, CHP)

    x_p = jnp.pad(x, ((0, NPAD - N_NODES), (0, 0)))
    b_p = jnp.pad(batch.reshape(N_NODES, 1), ((0, NPAD - N_NODES), (0, 0)),
                  constant_values=-1)

    degp = _sc_degree(dst2d).reshape(NC, HP)
    xw1 = _tc_matmul(x_p, W1)
    dpt = jnp.transpose(degp)
    y1, dinv = _tc_scale(xw1, dpt)

    agg1 = _sc_aggregate(y1, srcp, dstp)
    y2 = _tc_mid(agg1, y1, dinv, b1.reshape(1, D), W2)

    agg2 = _sc_aggregate(y2, srcp, dstp)
    pooled = _tc_pool(agg2, y2, dinv, b2.reshape(1, D), b_p)

    Wd_p = jnp.pad(Wd, ((0, 0), (0, NPAD - N_NODES)))
    bd_p = jnp.pad(bd.reshape(1, N_NODES), ((0, 0), (0, NPAD - N_NODES)))
    return _tc_dec(pooled, Wd_p, bd_p)[:, :N_NODES]


# speedup vs baseline: 3.7112x; 1.0773x over previous
import functools

import jax
import jax.numpy as jnp
from jax import lax
from jax.experimental import pallas as pl
from jax.experimental.pallas import tpu as pltpu
from jax.experimental.pallas import tpu_sc as plsc

N_NODES = 10000
D = 128
N_EDGES = 320000
N_GRAPHS = 64

NC = 2
NS = 16
NW = NC * NS
E_PER_TILE = N_EDGES // NW
CHUNK = 125
NCHUNK = E_PER_TILE // CHUNK
NPAD = 10240
ROWS_PER_TILE = NPAD // NS
ZROWS = 128
HP = 10240
HSLC = HP // NS

_mesh = plsc.VectorSubcoreMesh(core_axis_name="c", subcore_axis_name="s")

BLK = 1024
NBLK = NPAD // BLK


def _sc_degree(dst2d):
    @functools.partial(
        pl.kernel,
        out_type=jax.ShapeDtypeStruct((NC * HP,), jnp.float32),
        mesh=_mesh,
        scratch_types=[
            pltpu.VMEM((NCHUNK, CHUNK), jnp.int32),
            pltpu.VMEM((128,), jnp.float32),
            pltpu.VMEM((HSLC,), jnp.float32),
            pltpu.VMEM_SHARED((HP,), jnp.float32),
        ],
    )
    def k(dst_hbm, out_hbm, didx, ones_v, zv, shist):
        cid = lax.axis_index("c")
        sid = lax.axis_index("s")
        wid = cid * NS + sid

        @pl.loop(0, 128, step=16)
        def _(i):
            ones_v[pl.ds(i, 16)] = jnp.full((16,), 1.0, jnp.float32)

        @pl.loop(0, HSLC, step=16)
        def _(i):
            zv[pl.ds(i, 16)] = jnp.zeros((16,), jnp.float32)

        pltpu.sync_copy(zv, shist.at[pl.ds(sid * HSLC, HSLC)])
        pltpu.sync_copy(dst_hbm.at[pl.ds(wid * NCHUNK, NCHUNK)], didx)
        plsc.subcore_barrier()

        @pl.loop(0, NCHUNK)
        def _(j):
            pltpu.sync_copy(ones_v.at[pl.ds(0, CHUNK)],
                            shist.at[didx.at[j]], add=True)

        plsc.subcore_barrier()
        pltpu.sync_copy(shist.at[pl.ds(sid * HSLC, HSLC)],
                        out_hbm.at[pl.ds(cid * HP + sid * HSLC, HSLC)])

    return k(dst2d)


GRP = 8
NGRP = NCHUNK // GRP

EPAD = 327680
CHP = 80
NCHP = EPAD // NW // CHP
NGRPP = NCHP // GRP


def _sc_aggregate(y, srcp, dstp):
    @functools.partial(
        pl.kernel,
        out_type=jax.ShapeDtypeStruct((NC * NPAD, D), jnp.float32),
        mesh=_mesh,
        scratch_types=(
            [pltpu.VMEM((GRP, CHP), jnp.int32)] * 4
            + [pltpu.VMEM((GRP, CHP), jnp.int32)] * 4
            + [pltpu.VMEM((CHP, D), jnp.float32)] * 4
            + [pltpu.VMEM_SHARED((NPAD, D), jnp.float32)]
            + [pltpu.SemaphoreType.DMA] * 12
        ),
    )
    def k(y_hbm, src_hbm, dst_hbm, out_hbm,
          sr0, sr1, sr2, sr3, dr0, dr1, dr2, dr3,
          r0, r1, r2, r3, acc,
          i0, i1, i2, i3, g0, g1, g2, g3, s0, s1, s2, s3):
        cid = lax.axis_index("c")
        sid = lax.axis_index("s")
        wid = cid * NS + sid
        srings = (sr0, sr1, sr2, sr3)
        drings = (dr0, dr1, dr2, dr3)
        rows = (r0, r1, r2, r3)
        isems = (i0, i1, i2, i3)
        gsems = (g0, g1, g2, g3)
        ssems = (s0, s1, s2, s3)
        brow = wid * NCHP

        def idx_start(grp, sl):
            pltpu.async_copy(src_hbm.at[pl.ds(brow + grp * GRP, GRP)],
                             srings[sl], isems[sl])
            pltpu.async_copy(dst_hbm.at[pl.ds(brow + grp * GRP, GRP)],
                             drings[sl], isems[sl])

        def idx_wait(grp, sl):
            pltpu.make_async_copy(src_hbm.at[pl.ds(brow + grp * GRP, GRP)],
                                  srings[sl], isems[sl]).wait()
            pltpu.make_async_copy(dst_hbm.at[pl.ds(brow + grp * GRP, GRP)],
                                  drings[sl], isems[sl]).wait()

        def gather_start(sl, m, q):
            pltpu.async_copy(y_hbm.at[srings[sl].at[m]], rows[q], gsems[q])

        def gather_wait(sl, m, q):
            pltpu.make_async_copy(y_hbm.at[srings[sl].at[m]], rows[q],
                                  gsems[q]).wait()

        def scat_start(sl, m, q):
            pltpu.async_copy(rows[q], acc.at[drings[sl].at[m]], ssems[q],
                             add=True)

        def scat_wait(sl, m, q):
            pltpu.make_async_copy(rows[q], acc.at[drings[sl].at[m]],
                                  ssems[q]).wait()

        @pl.loop(0, CHP)
        def _(r):
            @pl.loop(0, D, step=16)
            def _(c):
                r0[r, pl.ds(c, 16)] = jnp.zeros((16,), jnp.float32)

        zbase = sid * ROWS_PER_TILE

        @pl.loop(0, ROWS_PER_TILE, step=CHP)
        def _(o):
            pltpu.async_copy(r0, acc.at[pl.ds(zbase + o, CHP)], g0)

        @pl.loop(0, ROWS_PER_TILE, step=CHP)
        def _(o):
            pltpu.make_async_copy(r0, acc.at[pl.ds(zbase + o, CHP)],
                                  g0).wait()

        plsc.subcore_barrier()

        idx_start(0, 0)
        idx_start(1, 1)
        idx_start(2, 2)
        idx_wait(0, 0)
        gather_start(0, 0, 0)
        gather_start(0, 1, 1)

        gather_start(0, 2, 2)

        def chunk_body(G, sl, m):
            q = m % 4
            t = G * GRP + m
            gather_wait(sl, m, q)
            scat_start(sl, m, q)

            if m >= 1:
                pv_sl, pv_m = sl, m - 1
            else:
                pv_sl, pv_m = (sl + 3) % 4, GRP - 1

            @pl.when(t > 0)
            def _():
                scat_wait(pv_sl, pv_m, (q + 3) % 4)

            if m == 2:
                @pl.when(G + 3 < NGRPP)
                def _():
                    idx_start(G + 3, (sl + 3) % 4)

            if m < GRP - 3:
                gather_start(sl, m + 3, (q + 3) % 4)
            elif m == GRP - 3:
                @pl.when(G + 1 < NGRPP)
                def _():
                    idx_wait(G + 1, (sl + 1) % 4)
                    gather_start((sl + 1) % 4, 0, (q + 3) % 4)
            else:
                @pl.when(G + 1 < NGRPP)
                def _():
                    gather_start((sl + 1) % 4, m - (GRP - 3), (q + 3) % 4)

        @pl.loop(0, NGRPP, step=4)
        def _(G):
            for k_ in range(4):
                for m in range(GRP):
                    chunk_body(G + k_, k_, m)

        scat_wait(3, GRP - 1, 3)
        plsc.subcore_barrier()

        @pl.loop(0, ROWS_PER_TILE, step=ZROWS)
        def _(o):
            pltpu.async_copy(acc.at[pl.ds(zbase + o, ZROWS)],
                             out_hbm.at[pl.ds(cid * NPAD + zbase + o, ZROWS)],
                             g0)

        @pl.loop(0, ROWS_PER_TILE, step=ZROWS)
        def _(o):
            pltpu.make_async_copy(acc.at[pl.ds(zbase + o, ZROWS)],
                                  out_hbm.at[pl.ds(cid * NPAD + zbase + o,
                                                   ZROWS)], g0).wait()

    return k(y, srcp, dstp)


def _tc_matmul(x, W):
    def body(x_ref, w_ref, o_ref):
        o_ref[...] = jnp.dot(x_ref[...], w_ref[...],
                             preferred_element_type=jnp.float32)

    return pl.pallas_call(
        body,
        grid=(NBLK,),
        in_specs=[pl.BlockSpec((BLK, D), lambda i: (i, 0)),
                  pl.BlockSpec((D, D), lambda i: (0, 0))],
        out_specs=pl.BlockSpec((BLK, D), lambda i: (i, 0)),
        out_shape=jax.ShapeDtypeStruct((NPAD, D), jnp.float32),
    )(x, W)


def _tc_scale(xw, dpt):
    def body(xw_ref, dp_ref, y_ref, dinv_ref):
        deg = 1.0 + dp_ref[:, 0:1] + dp_ref[:, 1:2]
        dinv = lax.rsqrt(deg)
        dinv_ref[...] = dinv
        y_ref[...] = xw_ref[...] * dinv

    return pl.pallas_call(
        body,
        grid=(NBLK,),
        in_specs=[pl.BlockSpec((BLK, D), lambda i: (i, 0)),
                  pl.BlockSpec((BLK, 2), lambda i: (i, 0))],
        out_specs=[pl.BlockSpec((BLK, D), lambda i: (i, 0)),
                   pl.BlockSpec((BLK, 1), lambda i: (i, 0))],
        out_shape=[jax.ShapeDtypeStruct((NPAD, D), jnp.float32),
                   jax.ShapeDtypeStruct((NPAD, 1), jnp.float32)],
    )(xw, dpt)


_A0 = pl.BlockSpec((BLK, D), lambda i: (i, 0))
_A1 = pl.BlockSpec((BLK, D), lambda i: (NBLK + i, 0))


def _tc_mid(agg, y1, dinv, b1, W2):
    def body(a0_ref, a1_ref, y1_ref, dinv_ref, b1_ref, w2_ref, y2_ref):
        dinv = dinv_ref[...]
        h = (a0_ref[...] + a1_ref[...] + y1_ref[...]) * dinv + b1_ref[...]
        h = jnp.maximum(h, 0.0)
        y2_ref[...] = jnp.dot(h, w2_ref[...],
                              preferred_element_type=jnp.float32) * dinv

    return pl.pallas_call(
        body,
        grid=(NBLK,),
        in_specs=[_A0, _A1,
                  pl.BlockSpec((BLK, D), lambda i: (i, 0)),
                  pl.BlockSpec((BLK, 1), lambda i: (i, 0)),
                  pl.BlockSpec((1, D), lambda i: (0, 0)),
                  pl.BlockSpec((D, D), lambda i: (0, 0))],
        out_specs=pl.BlockSpec((BLK, D), lambda i: (i, 0)),
        out_shape=jax.ShapeDtypeStruct((NPAD, D), jnp.float32),
    )(agg, agg, y1, dinv, b1, W2)


def _tc_pool(agg, y2, dinv, b2, bcol):
    def body(a0_ref, a1_ref, y2_ref, dinv_ref, b2_ref, b_ref, p_ref):
        i = pl.program_id(0)

        @pl.when(i == 0)
        def _():
            p_ref[...] = jnp.full((N_GRAPHS, D), -jnp.inf, jnp.float32)

        h = ((a0_ref[...] + a1_ref[...] + y2_ref[...]) * dinv_ref[...]
             + b2_ref[...])
        b = b_ref[...]
        lo = jnp.maximum(jnp.min(b), 0)
        hi = jnp.max(b)

        def upd(g, carry):
            m = jnp.max(jnp.where(b == g, h, -jnp.inf), axis=0, keepdims=True)
            p_ref[pl.ds(g, 1), :] = jnp.maximum(p_ref[pl.ds(g, 1), :], m)
            return carry

        lax.fori_loop(lo, hi + 1, upd, 0)

    return pl.pallas_call(
        body,
        grid=(NBLK,),
        in_specs=[_A0, _A1,
                  pl.BlockSpec((BLK, D), lambda i: (i, 0)),
                  pl.BlockSpec((BLK, 1), lambda i: (i, 0)),
                  pl.BlockSpec((1, D), lambda i: (0, 0)),
                  pl.BlockSpec((BLK, 1), lambda i: (i, 0))],
        out_specs=pl.BlockSpec((N_GRAPHS, D), lambda i: (0, 0)),
        out_shape=jax.ShapeDtypeStruct((N_GRAPHS, D), jnp.float32),
    )(agg, agg, y2, dinv, b2, bcol)


def _tc_dec(pooled, Wd, bd):
    NP = Wd.shape[1]
    CBLK = 1024

    def body(p_ref, wd_ref, bd_ref, o_ref):
        o_ref[...] = jnp.dot(p_ref[...], wd_ref[...],
                             preferred_element_type=jnp.float32) + bd_ref[...]

    return pl.pallas_call(
        body,
        grid=(NP // CBLK,),
        in_specs=[pl.BlockSpec((N_GRAPHS, D), lambda i: (0, 0)),
                  pl.BlockSpec((D, CBLK), lambda i: (0, i)),
                  pl.BlockSpec((1, CBLK), lambda i: (0, i))],
        out_specs=pl.BlockSpec((N_GRAPHS, CBLK), lambda i: (0, i)),
        out_shape=jax.ShapeDtypeStruct((N_GRAPHS, NP), jnp.float32),
    )(pooled, Wd, bd)


def kernel(x, edge_index, batch, W1, b1, W2, b2, Wd, bd):
    dst2d = edge_index[1].reshape(N_EDGES // CHUNK, CHUNK)
    pad_i = jnp.arange(EPAD - N_EDGES, dtype=jnp.int32)
    srcp = jnp.concatenate(
        [edge_index[0], pad_i % N_NODES]).reshape(EPAD // CHP, CHP)
    dstp = jnp.concatenate(
        [edge_index[1], N_NODES + pad_i % (NPAD - N_NODES)]
    ).reshape(EPAD // CHP, CHP)

    x_p = jnp.pad(x, ((0, NPAD - N_NODES), (0, 0)))
    b_p = jnp.pad(batch.reshape(N_NODES, 1), ((0, NPAD - N_NODES), (0, 0)),
                  constant_values=-1)

    degp = _sc_degree(dst2d).reshape(NC, HP)
    xw1 = _tc_matmul(x_p, W1)
    dpt = jnp.transpose(degp)
    y1, dinv = _tc_scale(xw1, dpt)

    agg1 = _sc_aggregate(y1, srcp, dstp)
    y2 = _tc_mid(agg1, y1, dinv, b1.reshape(1, D), W2)

    agg2 = _sc_aggregate(y2, srcp, dstp)
    pooled = _tc_pool(agg2, y2, dinv, b2.reshape(1, D), b_p)

    Wd_p = jnp.pad(Wd, ((0, 0), (0, NPAD - N_NODES)))
    bd_p = jnp.pad(bd.reshape(1, N_NODES), ((0, 0), (0, NPAD - N_NODES)))
    return _tc_dec(pooled, Wd_p, bd_p)[:, :N_NODES]
